# baseline (speedup 1.0000x reference)
import functools

import jax
import jax.numpy as jnp
from jax import lax
from jax.experimental import pallas as pl
from jax.experimental.pallas import tpu as pltpu
from jax.experimental.pallas import tpu_sc as plsc

N_EMBED = 1000000
HDIM = 64
BATCH = 16384
FIELDS = 26
N_TOTAL = BATCH * FIELDS

NW = 32
B_PER_W = N_TOTAL // NW
CHUNK = 512
N_CHUNKS = B_PER_W // CHUNK

_mesh = plsc.VectorSubcoreMesh(core_axis_name="c", subcore_axis_name="s")


@functools.partial(
    pl.kernel,
    mesh=_mesh,
    out_type=jax.ShapeDtypeStruct((N_TOTAL, HDIM), jnp.float32),
    scratch_types=[
        pltpu.VMEM((B_PER_W,), jnp.int32),
        pltpu.VMEM((CHUNK, HDIM), jnp.float32),
        pltpu.VMEM((CHUNK, HDIM), jnp.float32),
        pltpu.SemaphoreType.DMA,
        pltpu.SemaphoreType.DMA,
    ],
    compiler_params=pltpu.CompilerParams(use_tc_tiling_on_sc=False),
)
def _gather_kernel(idx_hbm, table_hbm, out_hbm, idx_v, buf0, buf1, sem0, sem1):
    wid = lax.axis_index("s") * 2 + lax.axis_index("c")
    base = wid * B_PER_W
    pltpu.sync_copy(idx_hbm.at[pl.ds(base, B_PER_W)], idx_v)

    bufs = (buf0, buf1)
    sems = (sem0, sem1)

    def gather_start(g, slot):
        idx_c = idx_v.at[pl.ds(g * CHUNK, CHUNK)]
        pltpu.async_copy(table_hbm.at[idx_c], bufs[slot], sems[slot])

    def store_out(g, slot):
        pltpu.sync_copy(bufs[slot], out_hbm.at[pl.ds(base + g * CHUNK, CHUNK)])

    gather_start(0, 0)

    def body(i, carry):
        g = i * 2

        @pl.when(g + 1 < N_CHUNKS)
        def _():
            gather_start(g + 1, 1)

        pltpu.make_async_copy(table_hbm.at[idx_v.at[pl.ds(0, CHUNK)]],
                              bufs[0], sems[0]).wait()
        store_out(g, 0)

        @pl.when(g + 2 < N_CHUNKS)
        def _():
            gather_start(g + 2, 0)

        @pl.when(g + 1 < N_CHUNKS)
        def _():
            pltpu.make_async_copy(table_hbm.at[idx_v.at[pl.ds(0, CHUNK)]],
                                  bufs[1], sems[1]).wait()
            store_out(g + 1, 1)

        return carry

    lax.fori_loop(0, (N_CHUNKS + 1) // 2, body, 0)


def kernel(indices, embs):
    idx_flat = indices.astype(jnp.int32).reshape(N_TOTAL)
    out = _gather_kernel(idx_flat, embs)
    return out.reshape(BATCH, FIELDS, HDIM)
